# baseline (speedup 1.0000x reference)
import functools

import jax
import jax.numpy as jnp
from jax import lax
from jax.experimental import pallas as pl
from jax.experimental.pallas import tpu as pltpu
from jax.experimental.pallas import tpu_sc as plsc

N = 512
IN_DIM = 128
HID = 256
ZD = 64
E = 16384
K = N * (N - 1) // 2
KPAD = 131072
PAD_K = 131008

EPT = E // 16
NCH = EPT // 128

MADJ = N * N
MCHUNK = MADJ // 16
YCHUNK = KPAD // 16
KCHUNK = K // 16


def _sc_body(edge_hbm, zeros_hbm, madj_out, y_out,
             src_v, dst_v, idx, ones_v, vbuf, acc_sh):
    c = lax.axis_index("c")
    s = lax.axis_index("s")
    base = s * EPT

    pltpu.sync_copy(zeros_hbm, vbuf)

    @pl.when(c == 0)
    def _():
        pltpu.sync_copy(vbuf.at[pl.ds(0, YCHUNK)], acc_sh.at[pl.ds(s * YCHUNK, YCHUNK)])

    @pl.when(c == 1)
    def _():
        pltpu.sync_copy(vbuf, acc_sh.at[pl.ds(s * MCHUNK, MCHUNK)])

    pltpu.sync_copy(edge_hbm.at[pl.ds(base, EPT)], src_v)
    pltpu.sync_copy(edge_hbm.at[pl.ds(E + base, EPT)], dst_v)

    for q in range(8):
        ones_v[pl.ds(q * 16, 16)] = jnp.full((16,), 1.0, jnp.float32)

    c_n = jnp.full((16,), N, jnp.int32)
    c_2nm1 = jnp.full((16,), 2 * N - 1, jnp.int32)
    c_one = jnp.full((16,), 1, jnp.int32)
    c_pad = jnp.full((16,), PAD_K, jnp.int32)

    @pl.when(c == 0)
    def _():
        for r in range(EPT // 16):
            sv = src_v[pl.ds(r * 16, 16)]
            dv = dst_v[pl.ds(r * 16, 16)]
            i_ = jnp.minimum(sv, dv)
            j_ = jnp.maximum(sv, dv)
            k = lax.shift_right_arithmetic(i_ * (c_2nm1 - i_), c_one) + j_ - i_ - c_one
            k = jnp.where(sv == dv, c_pad, k)
            idx[r // 8, pl.ds((r % 8) * 16, 16)] = k

    @pl.when(c == 1)
    def _():
        for r in range(EPT // 16):
            sv = src_v[pl.ds(r * 16, 16)]
            dv = dst_v[pl.ds(r * 16, 16)]
            idx[r // 8, pl.ds((r % 8) * 16, 16)] = dv * c_n + sv

    plsc.subcore_barrier()

    for q in range(NCH):
        pltpu.sync_copy(ones_v, acc_sh.at[idx.at[q]], add=True)

    plsc.subcore_barrier()

    @pl.when(c == 0)
    def _():
        pltpu.sync_copy(acc_sh.at[pl.ds(s * KCHUNK, KCHUNK)], vbuf.at[pl.ds(0, KCHUNK)])
        pltpu.sync_copy(vbuf.at[pl.ds(0, KCHUNK)], y_out.at[pl.ds(s * KCHUNK, KCHUNK)])

    @pl.when(c == 1)
    def _():
        pltpu.sync_copy(acc_sh.at[pl.ds(s * MCHUNK, MCHUNK)], vbuf)
        pltpu.sync_copy(vbuf, madj_out.at[pl.ds(s * MCHUNK, MCHUNK)])


@functools.cache
def _sc_build_fn():
    return pl.kernel(
        _sc_body,
        out_type=(
            jax.ShapeDtypeStruct((MADJ,), jnp.float32),
            jax.ShapeDtypeStruct((K,), jnp.float32),
        ),
        mesh=plsc.VectorSubcoreMesh(core_axis_name="c", subcore_axis_name="s"),
        scratch_types=[
            pltpu.VMEM((EPT,), jnp.int32),
            pltpu.VMEM((EPT,), jnp.int32),
            pltpu.VMEM((NCH, 128), jnp.int32),
            pltpu.VMEM((128,), jnp.float32),
            pltpu.VMEM((MCHUNK,), jnp.float32),
            pltpu.VMEM_SHARED((MADJ,), jnp.float32),
        ],
    )


def _dotT(a, b):
    return lax.dot_general(a, b, (((1,), (1,)), ((), ())),
                           preferred_element_type=jnp.float32,
                           precision=lax.Precision.HIGHEST)


def _encode(madj_ref, x_ref, w1_ref, b1_ref, g1_ref, be1_ref,
            w2_ref, b2_ref, g2_ref, be2_ref, wmu_ref, bmu_ref,
            wlv_ref, blv_ref, wd1_ref, bd1_ref, eps_ref,
            hdec_ref, kl_ref):
    madj = madj_ref[...]
    deg = 1.0 + jnp.sum(madj, axis=1, keepdims=True)
    dinv = lax.rsqrt(deg)

    def gcn_bn_relu(h, w, b, g, be):
        u = _dotT(h, w) * dinv
        agg = (lax.dot_general(madj, u, (((1,), (0,)), ((), ())),
                               preferred_element_type=jnp.float32,
                               precision=lax.Precision.HIGHEST) + u) * dinv
        hh = agg + b
        m = jnp.mean(hh, axis=0, keepdims=True)
        v = jnp.mean((hh - m) ** 2, axis=0, keepdims=True)
        return jnp.maximum((hh - m) * lax.rsqrt(v + 1e-5) * g + be, 0.0)

    h1 = gcn_bn_relu(x_ref[...], w1_ref[...], b1_ref[...], g1_ref[...], be1_ref[...])
    h2 = gcn_bn_relu(h1, w2_ref[...], b2_ref[...], g2_ref[...], be2_ref[...])

    gp = jnp.sum(h2, axis=0, keepdims=True) * (1.0 / N)
    mu = _dotT(gp, wmu_ref[...]) + bmu_ref[...]
    logvar = _dotT(gp, wlv_ref[...]) + blv_ref[...]
    z = mu + eps_ref[...] * jnp.exp(0.5 * logvar)
    hdec_ref[...] = jnp.maximum(_dotT(z, wd1_ref[...]) + bd1_ref[...], 0.0)
    klt = 1.0 + logvar - mu * mu - jnp.exp(logvar)
    kl_ref[...] = -0.5 / ZD * jnp.sum(klt, axis=(0, 1), keepdims=True)


def _encoder(madj2, x, W1, b1, g1, be1, W2, b2, g2, be2,
             Wmu, bmu, Wlv, blv, Wd1, bd1, eps):
    return pl.pallas_call(
        _encode,
        out_shape=(
            jax.ShapeDtypeStruct((1, HID), jnp.float32),
            jax.ShapeDtypeStruct((1, 1), jnp.float32),
        ),
    )(madj2, x, W1, b1[None, :], g1[None, :], be1[None, :],
      W2, b2[None, :], g2[None, :], be2[None, :],
      Wmu, bmu[None, :], Wlv, blv[None, :], Wd1, bd1[None, :], eps[None, :])


BLK = 11744
TCGRID = 8
TCROWS = TCGRID * BLK
SCROWS = K - TCROWS
RPT = SCROWS // 32
CHROWS = 128
NCHK = RPT // CHROWS
GPC = CHROWS // 16


def _loss_body(w_ref, b_ref, y_ref, h_ref, kl_ref, out_ref, acc_ref):
    i = pl.program_id(0)
    l = lax.dot_general(h_ref[...], w_ref[...], (((1,), (1,)), ((), ())),
                        preferred_element_type=jnp.float32) + b_ref[0]
    y = jnp.minimum(y_ref[0], 1.0)
    term = jnp.maximum(l, 0.0) + jnp.log1p(jnp.exp(-jnp.abs(l))) - y * l

    @pl.when(i == 0)
    def _():
        acc_ref[...] = jnp.zeros((1, BLK), jnp.float32)

    acc_ref[...] += term

    @pl.when(i == TCGRID - 1)
    def _():
        rec = jnp.sum(acc_ref[...], axis=(0, 1), keepdims=True) * (1.0 / K)
        out_ref[...] = rec + kl_ref[...]


def _loss(Wd2, bd2_tc, y_tc, hdec, kl):
    return pl.pallas_call(
        _loss_body,
        grid=(TCGRID,),
        in_specs=[
            pl.BlockSpec((BLK, HID), lambda i: (i, 0)),
            pl.BlockSpec((1, 1, BLK), lambda i: (i, 0, 0)),
            pl.BlockSpec((1, 1, BLK), lambda i: (i, 0, 0)),
            pl.BlockSpec((1, HID), lambda i: (0, 0)),
            pl.BlockSpec((1, 1), lambda i: (0, 0)),
        ],
        out_specs=pl.BlockSpec((1, 1), lambda i: (0, 0)),
        out_shape=jax.ShapeDtypeStruct((1, 1), jnp.float32),
        scratch_shapes=[pltpu.VMEM((1, BLK), jnp.float32)],
    )(Wd2, bd2_tc.reshape(TCGRID, 1, BLK), y_tc.reshape(TCGRID, 1, BLK),
      hdec, kl)


def _splat(vec16, j):
    idx = jnp.full((16, 1), j, jnp.int32)
    dn = lax.GatherDimensionNumbers(offset_dims=(), collapsed_slice_dims=(0,),
                                    start_index_map=(0,))
    return lax.gather(vec16, idx, dn, slice_sizes=(1,),
                      mode=lax.GatherScatterMode.PROMISE_IN_BOUNDS)


def _f32v(val):
    return jnp.full((16,), val, jnp.float32)


def _sc_tail_body(wd2_hbm, bs_hbm, ys_hbm, h_hbm, out_hbm,
                  hv, hsplat, yv, bv, w0, w1, accv, sem0, sem1):
    c = lax.axis_index("c")
    s = lax.axis_index("s")
    wid = s * 2 + c
    loc0 = pl.multiple_of(wid * RPT, 128)
    row0 = pl.multiple_of(TCROWS + wid * RPT, 128)

    pltpu.sync_copy(h_hbm, hv)
    pltpu.sync_copy(ys_hbm.at[pl.ds(loc0, RPT)], yv)
    pltpu.sync_copy(bs_hbm.at[pl.ds(loc0, RPT)], bv)

    bufs = (w0, w1)
    sems = (sem0, sem1)
    descs = [None, None]
    row0w = pl.multiple_of(row0 * HID, 128)
    descs[0] = pltpu.async_copy(wd2_hbm.at[pl.ds(row0w, CHROWS * HID)], w0, sem0)

    for cb in range(16):
        hc = hv[pl.ds(cb * 16, 16)]
        for j in range(16):
            hsplat[pl.ds((cb * 16 + j) * 16, 16)] = _splat(hc, j)

    iota = lax.broadcasted_iota(jnp.int32, (16,), 0)
    bases = [(iota + jnp.full((16,), g * 16, jnp.int32))
             * jnp.full((16,), HID, jnp.int32) for g in range(GPC)]
    zero = _f32v(0.0)
    one = _f32v(1.0)
    two = _f32v(2.0)
    c3, c5, c7, c9 = _f32v(1 / 3), _f32v(1 / 5), _f32v(1 / 7), _f32v(1 / 9)

    total = zero
    for ch in range(NCHK):
        if ch + 1 < NCHK:
            nxt = (ch + 1) % 2
            descs[nxt] = pltpu.async_copy(
                wd2_hbm.at[pl.ds(row0w + (ch + 1) * CHROWS * HID, CHROWS * HID)],
                bufs[nxt], sems[nxt])
        descs[ch % 2].wait()
        buf = bufs[ch % 2]

        def dot_step(it, accs, buf=buf):
            for df in range(8):
                f = it * 8 + df
                col = jnp.full((16,), f, jnp.int32)
                hs = hsplat[pl.ds(pl.multiple_of(f * 16, 16), 16)]
                accs = tuple(
                    accs[g] + plsc.load_gather(buf, [bases[g] + col]) * hs
                    for g in range(GPC))
            return accs

        accs = lax.fori_loop(0, HID // 8, dot_step, (zero,) * GPC)

        for g in range(GPC):
            gg = ch * GPC + g
            l = accs[g] + bv[pl.ds(gg * 16, 16)]
            y = jnp.minimum(yv[pl.ds(gg * 16, 16)], one)
            e = jnp.exp(-jnp.abs(l))
            u = e / (e + two)
            u2 = u * u
            log1p = (two * u) * (one + u2 * (c3 + u2 * (c5 + u2 * (c7 + u2 * c9))))
            total = total + jnp.maximum(l, zero) + log1p - y * l

    accv[...] = total
    pltpu.sync_copy(accv, out_hbm.at[wid])


@functools.cache
def _sc_tail_fn():
    return pl.kernel(
        _sc_tail_body,
        out_type=jax.ShapeDtypeStruct((32, 16), jnp.float32),
        mesh=plsc.VectorSubcoreMesh(core_axis_name="c", subcore_axis_name="s"),
        compiler_params=pltpu.CompilerParams(needs_layout_passes=False),
        scratch_types=[
            pltpu.VMEM((HID,), jnp.float32),
            pltpu.VMEM((HID * 16,), jnp.float32),
            pltpu.VMEM((RPT,), jnp.float32),
            pltpu.VMEM((RPT,), jnp.float32),
            pltpu.VMEM((CHROWS * HID,), jnp.float32),
            pltpu.VMEM((CHROWS * HID,), jnp.float32),
            pltpu.VMEM((16,), jnp.float32),
            pltpu.SemaphoreType.DMA,
            pltpu.SemaphoreType.DMA,
        ],
    )


def kernel(x, edge_index, eps, W1, b1, gamma1, beta1, W2, b2, gamma2, beta2,
           Wmu, bmu, Wlv, blv, Wd1, bd1, Wd2, bd2):
    edge_flat = edge_index.reshape(-1)
    zeros = jnp.zeros((MCHUNK,), jnp.float32)
    madj_f, y = _sc_build_fn()(edge_flat, zeros)
    hdec, kl = _encoder(madj_f.reshape(N, N), x, W1, b1, gamma1, beta1,
                        W2, b2, gamma2, beta2, Wmu, bmu, Wlv, blv, Wd1, bd1, eps)
    tail = _sc_tail_fn()(Wd2.reshape(K * HID), bd2[TCROWS:], y[TCROWS:],
                         hdec.reshape(HID))
    out = _loss(Wd2, bd2[:TCROWS], y[:TCROWS], hdec, kl)
    return out[0, 0] + jnp.sum(tail) * (1.0 / K)


# speedup vs baseline: 3.5097x; 3.5097x over previous
import functools

import jax
import jax.numpy as jnp
from jax import lax
from jax.experimental import pallas as pl
from jax.experimental.pallas import tpu as pltpu
from jax.experimental.pallas import tpu_sc as plsc

N = 512
IN_DIM = 128
HID = 256
ZD = 64
E = 16384
K = N * (N - 1) // 2
KPAD = 131072
PAD_K = 131008

EPT = E // 16
NCH = EPT // 128

MADJ = N * N
MCHUNK = MADJ // 16
YCHUNK = KPAD // 16
KCHUNK = K // 16


def _sc_body(edge_hbm, zeros_hbm, madj_out, y_out,
             src_v, dst_v, idx, ones_v, vbuf, acc_sh):
    c = lax.axis_index("c")
    s = lax.axis_index("s")
    base = s * EPT

    pltpu.sync_copy(zeros_hbm, vbuf)

    @pl.when(c == 0)
    def _():
        pltpu.sync_copy(vbuf.at[pl.ds(0, YCHUNK)], acc_sh.at[pl.ds(s * YCHUNK, YCHUNK)])

    @pl.when(c == 1)
    def _():
        pltpu.sync_copy(vbuf, acc_sh.at[pl.ds(s * MCHUNK, MCHUNK)])

    pltpu.sync_copy(edge_hbm.at[pl.ds(base, EPT)], src_v)
    pltpu.sync_copy(edge_hbm.at[pl.ds(E + base, EPT)], dst_v)

    for q in range(8):
        ones_v[pl.ds(q * 16, 16)] = jnp.full((16,), 1.0, jnp.float32)

    c_n = jnp.full((16,), N, jnp.int32)
    c_2nm1 = jnp.full((16,), 2 * N - 1, jnp.int32)
    c_one = jnp.full((16,), 1, jnp.int32)
    c_pad = jnp.full((16,), PAD_K, jnp.int32)

    @pl.when(c == 0)
    def _():
        for r in range(EPT // 16):
            sv = src_v[pl.ds(r * 16, 16)]
            dv = dst_v[pl.ds(r * 16, 16)]
            i_ = jnp.minimum(sv, dv)
            j_ = jnp.maximum(sv, dv)
            k = lax.shift_right_arithmetic(i_ * (c_2nm1 - i_), c_one) + j_ - i_ - c_one
            k = jnp.where(sv == dv, c_pad, k)
            idx[r // 8, pl.ds((r % 8) * 16, 16)] = k

    @pl.when(c == 1)
    def _():
        for r in range(EPT // 16):
            sv = src_v[pl.ds(r * 16, 16)]
            dv = dst_v[pl.ds(r * 16, 16)]
            idx[r // 8, pl.ds((r % 8) * 16, 16)] = dv * c_n + sv

    plsc.subcore_barrier()

    for q in range(NCH):
        pltpu.sync_copy(ones_v, acc_sh.at[idx.at[q]], add=True)

    plsc.subcore_barrier()

    @pl.when(c == 0)
    def _():
        pltpu.sync_copy(acc_sh.at[pl.ds(s * KCHUNK, KCHUNK)], vbuf.at[pl.ds(0, KCHUNK)])
        pltpu.sync_copy(vbuf.at[pl.ds(0, KCHUNK)], y_out.at[pl.ds(s * KCHUNK, KCHUNK)])

    @pl.when(c == 1)
    def _():
        pltpu.sync_copy(acc_sh.at[pl.ds(s * MCHUNK, MCHUNK)], vbuf)
        pltpu.sync_copy(vbuf, madj_out.at[pl.ds(s * MCHUNK, MCHUNK)])


@functools.cache
def _sc_build_fn():
    return pl.kernel(
        _sc_body,
        out_type=(
            jax.ShapeDtypeStruct((MADJ,), jnp.float32),
            jax.ShapeDtypeStruct((K,), jnp.float32),
        ),
        mesh=plsc.VectorSubcoreMesh(core_axis_name="c", subcore_axis_name="s"),
        scratch_types=[
            pltpu.VMEM((EPT,), jnp.int32),
            pltpu.VMEM((EPT,), jnp.int32),
            pltpu.VMEM((NCH, 128), jnp.int32),
            pltpu.VMEM((128,), jnp.float32),
            pltpu.VMEM((MCHUNK,), jnp.float32),
            pltpu.VMEM_SHARED((MADJ,), jnp.float32),
        ],
    )


def _dotT(a, b):
    return lax.dot_general(a, b, (((1,), (1,)), ((), ())),
                           preferred_element_type=jnp.float32,
                           precision=lax.Precision.HIGHEST)


def _encode(madj_ref, x_ref, w1_ref, b1_ref, g1_ref, be1_ref,
            w2_ref, b2_ref, g2_ref, be2_ref, wmu_ref, bmu_ref,
            wlv_ref, blv_ref, wd1_ref, bd1_ref, eps_ref,
            hdec_ref, kl_ref):
    madj = madj_ref[...]
    deg = 1.0 + jnp.sum(madj, axis=1, keepdims=True)
    dinv = lax.rsqrt(deg)

    def gcn_bn_relu(h, w, b, g, be):
        u = _dotT(h, w) * dinv
        agg = (lax.dot_general(madj, u, (((1,), (0,)), ((), ())),
                               preferred_element_type=jnp.float32,
                               precision=lax.Precision.HIGHEST) + u) * dinv
        hh = agg + b
        m = jnp.mean(hh, axis=0, keepdims=True)
        v = jnp.mean((hh - m) ** 2, axis=0, keepdims=True)
        return jnp.maximum((hh - m) * lax.rsqrt(v + 1e-5) * g + be, 0.0)

    h1 = gcn_bn_relu(x_ref[...], w1_ref[...], b1_ref[...], g1_ref[...], be1_ref[...])
    h2 = gcn_bn_relu(h1, w2_ref[...], b2_ref[...], g2_ref[...], be2_ref[...])

    gp = jnp.sum(h2, axis=0, keepdims=True) * (1.0 / N)
    mu = _dotT(gp, wmu_ref[...]) + bmu_ref[...]
    logvar = _dotT(gp, wlv_ref[...]) + blv_ref[...]
    z = mu + eps_ref[...] * jnp.exp(0.5 * logvar)
    hdec_ref[...] = jnp.maximum(_dotT(z, wd1_ref[...]) + bd1_ref[...], 0.0)
    klt = 1.0 + logvar - mu * mu - jnp.exp(logvar)
    kl_ref[...] = -0.5 / ZD * jnp.sum(klt, axis=(0, 1), keepdims=True)


BLK = 18688
HBLK = BLK // 2
GRID = K // BLK


def _fused_body(wa_ref, wb_ref, b_ref, y_ref, madj_ref, x_ref,
                w1_ref, b1_ref, g1_ref, be1_ref, w2_ref, b2_ref, g2_ref,
                be2_ref, wmu_ref, bmu_ref, wlv_ref, blv_ref, wd1_ref,
                bd1_ref, eps_ref, out_ref, acc_ref, hdec_s, kl_s):
    i = pl.program_id(0)

    @pl.when(i == 0)
    def _():
        _encode(madj_ref, x_ref, w1_ref, b1_ref, g1_ref, be1_ref,
                w2_ref, b2_ref, g2_ref, be2_ref, wmu_ref, bmu_ref,
                wlv_ref, blv_ref, wd1_ref, bd1_ref, eps_ref, hdec_s, kl_s)
        acc_ref[...] = jnp.zeros((1, BLK), jnp.float32)

    h = hdec_s[...]
    b = b_ref[0]
    yc = jnp.minimum(y_ref[0], 1.0)
    for half, w_ref in ((0, wa_ref), (1, wb_ref)):
        sl = (slice(None), slice(half * HBLK, (half + 1) * HBLK))
        l = lax.dot_general(h, w_ref[...], (((1,), (1,)), ((), ())),
                            preferred_element_type=jnp.float32) + b[sl]
        y = yc[sl]
        term = jnp.maximum(l, 0.0) + jnp.log1p(jnp.exp(-jnp.abs(l))) - y * l
        acc_ref[:, pl.ds(half * HBLK, HBLK)] += term

    @pl.when(i == GRID - 1)
    def _():
        rec = jnp.sum(acc_ref[...], axis=(0, 1), keepdims=True) * (1.0 / K)
        out_ref[...] = rec + kl_s[...]


def _fused(Wd2, bd2, y, madj2, x, W1, b1, g1, be1, W2, b2, g2, be2,
           Wmu, bmu, Wlv, blv, Wd1, bd1, eps):
    full = lambda shape: pl.BlockSpec(shape, lambda i: tuple(0 for _ in shape))
    return pl.pallas_call(
        _fused_body,
        grid=(GRID,),
        in_specs=[
            pl.BlockSpec((HBLK, HID), lambda i: (2 * i, 0)),
            pl.BlockSpec((HBLK, HID), lambda i: (2 * i + 1, 0)),
            pl.BlockSpec((1, 1, BLK), lambda i: (i, 0, 0)),
            pl.BlockSpec((1, 1, BLK), lambda i: (i, 0, 0)),
            full((N, N)), full((N, IN_DIM)),
            full((HID, IN_DIM)), full((1, HID)), full((1, HID)), full((1, HID)),
            full((HID, HID)), full((1, HID)), full((1, HID)), full((1, HID)),
            full((ZD, HID)), full((1, ZD)), full((ZD, HID)), full((1, ZD)),
            full((HID, ZD)), full((1, HID)), full((1, ZD)),
        ],
        out_specs=pl.BlockSpec((1, 1), lambda i: (0, 0)),
        out_shape=jax.ShapeDtypeStruct((1, 1), jnp.float32),
        scratch_shapes=[
            pltpu.VMEM((1, BLK), jnp.float32),
            pltpu.VMEM((1, HID), jnp.float32),
            pltpu.VMEM((1, 1), jnp.float32),
        ],
    )(Wd2, Wd2, bd2.reshape(GRID, 1, BLK), y.reshape(GRID, 1, BLK),
      madj2, x, W1, b1[None, :], g1[None, :], be1[None, :],
      W2, b2[None, :], g2[None, :], be2[None, :],
      Wmu, bmu[None, :], Wlv, blv[None, :], Wd1, bd1[None, :], eps[None, :])


def kernel(x, edge_index, eps, W1, b1, gamma1, beta1, W2, b2, gamma2, beta2,
           Wmu, bmu, Wlv, blv, Wd1, bd1, Wd2, bd2):
    edge_flat = edge_index.reshape(-1)
    zeros = jnp.zeros((MCHUNK,), jnp.float32)
    madj_f, y = _sc_build_fn()(edge_flat, zeros)
    out = _fused(Wd2, bd2, y, madj_f.reshape(N, N), x, W1, b1, gamma1, beta1,
                 W2, b2, gamma2, beta2, Wmu, bmu, Wlv, blv, Wd1, bd1, eps)
    return out[0, 0]
